# baseline (speedup 1.0000x reference)
import functools

import jax
import jax.numpy as jnp
from jax import lax
from jax.experimental import pallas as pl
from jax.experimental.pallas import tpu as pltpu
from jax.experimental.pallas import tpu_sc as plsc

VOCAB = 1000000
DIM = 32
BATCH = 16384
NEG = 5

_INFO = plsc.get_sparse_core_info()
NUM_CORES = _INFO.num_cores
NUM_SUBCORES = _INFO.num_subcores
LANES = _INFO.num_lanes
NW = NUM_CORES * NUM_SUBCORES
CHUNK = BATCH // NW
GCHUNK = 128
NT_CH = CHUNK // GCHUNK
NN = CHUNK * NEG
NN_CH = NN // GCHUNK
NGROUP = CHUNK // LANES


def _body(tw, cw, nw, emb_in, emb_out, pos_out, neg_out,
          idx_t, idx_c, idx_n, rows_t, rows_c, rows_n, pos_v, neg_v, sem):
    wid = lax.axis_index("s") * NUM_CORES + lax.axis_index("c")
    base = wid * CHUNK

    pltpu.sync_copy(tw.at[pl.ds(base, CHUNK)], idx_t)
    pltpu.sync_copy(cw.at[pl.ds(base, CHUNK)], idx_c)
    pltpu.sync_copy(nw.at[pl.ds(base * NEG, NN)], idx_n)

    copies = []
    for ci in range(NT_CH):
        s = ci * GCHUNK
        copies.append(pltpu.async_copy(
            emb_in.at[idx_t.at[pl.ds(s, GCHUNK)]],
            rows_t.at[pl.ds(s, GCHUNK), :], sem))
        copies.append(pltpu.async_copy(
            emb_out.at[idx_c.at[pl.ds(s, GCHUNK)]],
            rows_c.at[pl.ds(s, GCHUNK), :], sem))
    for ci in range(NN_CH):
        s = ci * GCHUNK
        copies.append(pltpu.async_copy(
            emb_out.at[idx_n.at[pl.ds(s, GCHUNK)]],
            rows_n.at[pl.ds(s, GCHUNK), :], sem))
    for c in copies:
        c.wait()

    def group(g, carry):
        row16 = lax.iota(jnp.int32, 16) + g * LANES
        nrows = [row16 * NEG + j for j in range(NEG)]
        accp = jnp.zeros((16,), jnp.float32)
        accn = [jnp.zeros((16,), jnp.float32) for _ in range(NEG)]
        for d in range(DIM):
            cold = jnp.full((16,), d, jnp.int32)
            t = plsc.load_gather(rows_t, [row16, cold])
            c = plsc.load_gather(rows_c, [row16, cold])
            accp = accp + t * c
            for j in range(NEG):
                nv = plsc.load_gather(rows_n, [nrows[j], cold])
                accn[j] = accn[j] + t * nv
        pos_v[pl.ds(g * LANES, 16)] = accp
        for j in range(NEG):
            plsc.store_scatter(
                neg_v, [row16, jnp.full((16,), j, jnp.int32)], accn[j])
        return carry

    lax.fori_loop(0, NGROUP, group, 0)

    pltpu.sync_copy(pos_v, pos_out.at[pl.ds(base, CHUNK)])
    pltpu.sync_copy(neg_v, neg_out.at[pl.ds(base, CHUNK)])


@jax.jit
def _run(tw, cw, nwf, emb_in, emb_out):
    f = pl.kernel(
        _body,
        out_type=[
            jax.ShapeDtypeStruct((BATCH,), jnp.float32),
            jax.ShapeDtypeStruct((BATCH, NEG), jnp.float32),
        ],
        mesh=plsc.VectorSubcoreMesh(core_axis_name="c", subcore_axis_name="s"),
        compiler_params=pltpu.CompilerParams(
            needs_layout_passes=False, use_tc_tiling_on_sc=False),
        scratch_types=[
            pltpu.VMEM((CHUNK,), jnp.int32),
            pltpu.VMEM((CHUNK,), jnp.int32),
            pltpu.VMEM((NN,), jnp.int32),
            pltpu.VMEM((CHUNK, DIM), jnp.float32),
            pltpu.VMEM((CHUNK, DIM), jnp.float32),
            pltpu.VMEM((NN, DIM), jnp.float32),
            pltpu.VMEM((CHUNK,), jnp.float32),
            pltpu.VMEM((CHUNK, NEG), jnp.float32),
            pltpu.SemaphoreType.DMA,
        ],
    )
    return f(tw, cw, nwf, emb_in, emb_out)


def kernel(target_word, context_word, negative_words, emb_in, emb_out):
    tw = target_word.astype(jnp.int32)
    cw = context_word.astype(jnp.int32)
    nwf = negative_words.astype(jnp.int32).reshape(BATCH * NEG)
    pos, neg = _run(tw, cw, nwf, emb_in, emb_out)
    return (pos, neg)
